# Initial kernel scaffold; baseline (speedup 1.0000x reference)
import jax
import jax.numpy as jnp
from jax.experimental import pallas as pl


def kernel(inputs, state, edge_index, W_gat, att_src, att_dst, b_gat, bias1, W1, b1, W2, b2):
    raise NotImplementedError("write your pallas kernel here")


# speedup vs baseline: 9.8852x; 9.8852x over previous
import jax
import jax.numpy as jnp
from jax import lax
from jax.experimental import pallas as pl
from jax.experimental.pallas import tpu as pltpu

_N = 240


def _step(sid_ref, did_ref, inp_ref, st_ref, wgs_ref, wgi_ref, asrc_ref,
          adst_ref, bg_ref, w1ir_ref, w1hr_ref, w1iu_ref, w1hu_ref,
          b1r_ref, b1u_ref, w2i_ref, w2h_ref, b2_ref, out_ref, m_scr):
    b = pl.program_id(0)
    f32 = jnp.float32

    @pl.when(b == 0)
    def _build_m():
        ep = sid_ref.shape[0]
        iota_n = lax.broadcasted_iota(jnp.int32, (ep, _N), 1)
        oh_s = (sid_ref[...] == iota_n).astype(f32)
        oh_d = (did_ref[...] == iota_n).astype(f32)
        m = lax.dot_general(oh_d, oh_s, (((0,), (0,)), ((), ())),
                            preferred_element_type=f32)
        row = lax.broadcasted_iota(jnp.int32, (_N, _N), 0)
        col = lax.broadcasted_iota(jnp.int32, (_N, _N), 1)
        m_scr[...] = m + (row == col).astype(f32)

    inp = inp_ref[0]
    st = st_ref[0]
    h = (jnp.dot(st, wgs_ref[...], preferred_element_type=f32)
         + jnp.dot(inp, wgi_ref[...], preferred_element_type=f32))
    a_d = jnp.dot(h, adst_ref[...], preferred_element_type=f32)
    a_s = lax.dot_general(asrc_ref[...], h, (((1,), (1,)), ((), ())),
                          preferred_element_type=f32)
    e = a_d + a_s
    e = jnp.where(e >= 0, e, 0.2 * e)
    m = m_scr[...]
    mask = m > 0
    em = jnp.where(mask, e, -1e30)
    rowmax = jnp.max(em, axis=1, keepdims=True)
    p = m * jnp.exp(em - rowmax)
    denom = jnp.sum(p, axis=1, keepdims=True)
    alpha = p / (denom + 1e-16)
    s2 = jnp.dot(alpha, h, preferred_element_type=f32) + bg_ref[...]
    r = jax.nn.sigmoid(jnp.dot(inp, w1ir_ref[...], preferred_element_type=f32)
                       + jnp.dot(s2, w1hr_ref[...], preferred_element_type=f32)
                       + b1r_ref[...])
    u = jax.nn.sigmoid(jnp.dot(inp, w1iu_ref[...], preferred_element_type=f32)
                       + jnp.dot(s2, w1hu_ref[...], preferred_element_type=f32)
                       + b1u_ref[...])
    c = jnp.tanh(jnp.dot(inp, w2i_ref[...], preferred_element_type=f32)
                 + jnp.dot(r * s2, w2h_ref[...], preferred_element_type=f32)
                 + b2_ref[...])
    out_ref[0] = u * s2 + (1.0 - u) * c


def kernel(inputs, state, edge_index, W_gat, att_src, att_dst, b_gat, bias1,
           W1, b1, W2, b2):
    B = inputs.shape[0]
    U = att_src.shape[0]
    F = W_gat.shape[0] - U
    N = _N
    ne = edge_index.shape[1]

    inp3 = inputs.reshape(B, N, F)
    st3 = state.reshape(B, N, U)
    ep = ((ne + 7) // 8) * 8
    pad = ((0, ep - ne),)
    sid = jnp.pad(edge_index[0], pad, constant_values=-1).reshape(ep, 1)
    did = jnp.pad(edge_index[1], pad, constant_values=-1).reshape(ep, 1)

    wgs = W_gat[:U]
    wgi = W_gat[U:]
    asrc = att_src.reshape(1, U)
    adst = att_dst.reshape(U, 1)
    bg = (b_gat + bias1).reshape(1, U)
    w1ir, w1iu = W1[:F, :U], W1[:F, U:]
    w1hr, w1hu = W1[F:, :U], W1[F:, U:]
    b1r, b1u = b1[:U].reshape(1, U), b1[U:].reshape(1, U)
    w2i, w2h = W2[:F], W2[F:]
    b2r = b2.reshape(1, U)

    def cmap(*shape):
        return pl.BlockSpec(shape, lambda b: (0,) * len(shape))

    out = pl.pallas_call(
        _step,
        grid=(B,),
        in_specs=[
            cmap(ep, 1), cmap(ep, 1),
            pl.BlockSpec((1, N, F), lambda b: (b, 0, 0)),
            pl.BlockSpec((1, N, U), lambda b: (b, 0, 0)),
            cmap(U, U), cmap(F, U), cmap(1, U), cmap(U, 1), cmap(1, U),
            cmap(F, U), cmap(U, U), cmap(F, U), cmap(U, U),
            cmap(1, U), cmap(1, U), cmap(F, U), cmap(U, U), cmap(1, U),
        ],
        out_specs=pl.BlockSpec((1, N, U), lambda b: (b, 0, 0)),
        out_shape=jax.ShapeDtypeStruct((B, N, U), jnp.float32),
        scratch_shapes=[pltpu.VMEM((N, N), jnp.float32)],
    )(sid, did, inp3, st3, wgs, wgi, asrc, adst, bg,
      w1ir, w1hr, w1iu, w1hu, b1r, b1u, w2i, w2h, b2r)
    return out.reshape(B, N * U)
